# Initial kernel scaffold; baseline (speedup 1.0000x reference)
import jax
import jax.numpy as jnp
from jax.experimental import pallas as pl


def kernel(patch_feats, patch_edge_weights, mesh_edge_weights, params, patch_edge_index, mesh_edge_index):
    raise NotImplementedError("write your pallas kernel here")


# speedup vs baseline: 1.4088x; 1.4088x over previous
import functools

import jax
import jax.numpy as jnp
from jax import lax
from jax.experimental import pallas as pl
from jax.experimental.pallas import tpu as pltpu

NEG = 0.01
_EPS = 1e-5


def _leaky(x):
    return jnp.where(x >= 0, x, NEG * x)


def _mm_kernel_body(refs, *, epi, has_w, has_aux, has_scale, has_stats, bn,
                    nrows_total, post_scale):
    idx = 0
    a_ref = refs[idx]; idx += 1
    w_ref = None
    if has_w:
        w_ref = refs[idx]; idx += 1
    aux_ref = None
    if has_aux:
        aux_ref = refs[idx]; idx += 1
    scale_ref = None
    if has_scale:
        scale_ref = refs[idx]; idx += 1
    stats_ref = None
    if has_stats:
        stats_ref = refs[idx]; idx += 1
    o_ref = refs[idx]

    a = a_ref[...]
    if has_scale:
        s = lax.rsqrt(jnp.maximum(scale_ref[...], 1.0))
        a = a * s
    if has_w:
        y = jnp.dot(a, w_ref[...], preferred_element_type=jnp.float32)
    else:
        y = a

    if epi == 'none':
        o_ref[...] = y
    elif epi == 'bias_leaky':
        y = _leaky(y + aux_ref[0, :][None, :])
        o_ref[...] = y
    elif epi == 'gn':
        m = y.shape[1]
        al = aux_ref[1, :][None, :]
        ga = aux_ref[2, :][None, :]
        be = aux_ref[3, :][None, :]
        r = y.reshape(bn // 10, 10, m)
        mean = jnp.mean(r, axis=1)
        xc = r - (al * mean)[:, None, :]
        var = jnp.mean(xc * xc, axis=1)
        out = _leaky(ga[:, None, :] * xc / jnp.sqrt(var[:, None, :] + _EPS)
                     + be[:, None, :])
        o_ref[...] = out.reshape(bn, m)
    elif epi == 'gn_global':
        n = float(nrows_total)
        s1 = stats_ref[0, :][None, :]
        s2 = stats_ref[1, :][None, :]
        mean = s1 / n
        ex2 = s2 / n
        al = aux_ref[1, :][None, :]
        ga = aux_ref[2, :][None, :]
        be = aux_ref[3, :][None, :]
        var = ex2 - 2.0 * al * mean * mean + al * al * mean * mean
        xc = y - al * mean
        o_ref[...] = _leaky(ga * xc / jnp.sqrt(var + _EPS) + be)
    elif epi == 'inorm':
        m = jnp.mean(y, axis=1, keepdims=True)
        v = jnp.mean((y - m) * (y - m), axis=1, keepdims=True)
        o_ref[...] = _leaky((y - m) / jnp.sqrt(v + _EPS))
    elif epi == 'segsum10':
        m = y.shape[1]
        r = y.reshape(bn // 10, 10, m).sum(axis=1)
        if post_scale != 1.0:
            r = r * post_scale
        o_ref[...] = r
    elif epi == 'bias_leaky_segsum10':
        m = y.shape[1]
        y = _leaky(y + aux_ref[0, :][None, :])
        r = y.reshape(bn // 10, 10, m).sum(axis=1)
        if post_scale != 1.0:
            r = r * post_scale
        o_ref[...] = r
    elif epi == 'colstats':
        m = y.shape[1]
        s1 = jnp.sum(y, axis=0, keepdims=True)
        s2 = jnp.sum(y * y, axis=0, keepdims=True)
        blk = jnp.concatenate([s1, s2, jnp.zeros((6, m), jnp.float32)], axis=0)

        @pl.when(pl.program_id(0) == 0)
        def _():
            o_ref[...] = blk

        @pl.when(pl.program_id(0) != 0)
        def _():
            o_ref[...] = o_ref[...] + blk
    elif epi == 'bias_leaky_colsum':
        m = y.shape[1]
        y = _leaky(y + aux_ref[0, :][None, :])
        s1 = jnp.sum(y, axis=0, keepdims=True)
        blk = jnp.concatenate([s1, jnp.zeros((7, m), jnp.float32)], axis=0)

        @pl.when(pl.program_id(0) == 0)
        def _():
            o_ref[...] = blk

        @pl.when(pl.program_id(0) != 0)
        def _():
            o_ref[...] = o_ref[...] + blk
    else:
        raise ValueError(epi)


def _mm(A, W=None, aux=None, scale=None, stats=None, *, epi='none',
        bn=2000, post_scale=1.0):
    n, k = A.shape
    assert n % bn == 0, (n, bn)
    m = W.shape[1] if W is not None else k
    grid = (n // bn,)

    in_specs = [pl.BlockSpec((bn, k), lambda i: (i, 0))]
    args = [A]
    if W is not None:
        in_specs.append(pl.BlockSpec((k, m), lambda i: (0, 0)))
        args.append(W)
    if aux is not None:
        in_specs.append(pl.BlockSpec((8, m), lambda i: (0, 0)))
        args.append(aux)
    if scale is not None:
        in_specs.append(pl.BlockSpec((bn, 1), lambda i: (i, 0)))
        args.append(scale.reshape(n, 1))
    if stats is not None:
        in_specs.append(pl.BlockSpec((8, m), lambda i: (0, 0)))
        args.append(stats)

    if epi in ('segsum10', 'bias_leaky_segsum10'):
        out_shape = jax.ShapeDtypeStruct((n // 10, m), jnp.float32)
        out_spec = pl.BlockSpec((bn // 10, m), lambda i: (i, 0))
    elif epi in ('colstats', 'bias_leaky_colsum'):
        out_shape = jax.ShapeDtypeStruct((8, m), jnp.float32)
        out_spec = pl.BlockSpec((8, m), lambda i: (0, 0))
    else:
        out_shape = jax.ShapeDtypeStruct((n, m), jnp.float32)
        out_spec = pl.BlockSpec((bn, m), lambda i: (i, 0))

    body = functools.partial(
        _mm_kernel_body, epi=epi, has_w=W is not None, has_aux=aux is not None,
        has_scale=scale is not None, has_stats=stats is not None, bn=bn,
        nrows_total=n, post_scale=post_scale)

    def kern(*refs):
        _mm_kernel_body_wrapper(refs, body)

    return pl.pallas_call(
        kern,
        grid=grid,
        in_specs=in_specs,
        out_specs=out_spec,
        out_shape=out_shape,
    )(*args)


def _mm_kernel_body_wrapper(refs, body):
    body(refs)


def _aux_pack(m, bias=None, alpha=None, gamma=None, beta=None):
    rows = []
    for v in (bias, alpha, gamma, beta):
        rows.append(jnp.zeros((m,), jnp.float32) if v is None else v)
    rows += [jnp.zeros((m,), jnp.float32)] * 4
    return jnp.stack(rows, axis=0)


def _tail_kernel(s1_ref, s2_ref, w1_ref, b1_ref, w2_ref, b2_ref, cls_ref,
                 o_ref):
    r1 = _leaky(jnp.dot(s1_ref[...], w1_ref[...],
                        preferred_element_type=jnp.float32) + b1_ref[0, :][None, :])
    r2 = _leaky(jnp.dot(s2_ref[...], w2_ref[...],
                        preferred_element_type=jnp.float32) + b2_ref[0, :][None, :])
    cat = jnp.concatenate([r1, r2], axis=1)
    o_ref[...] = jnp.dot(cat, cls_ref[...], preferred_element_type=jnp.float32)


def _tail(s1, s2, w1, b1, w2, b2, cls):
    kq = w1.shape[0]
    mq = w1.shape[1]
    out_n = cls.shape[1]
    return pl.pallas_call(
        _tail_kernel,
        out_shape=jax.ShapeDtypeStruct((8, out_n), jnp.float32),
    )(s1, s2, w1, _aux_pack(mq, bias=b1), w2, _aux_pack(mq, bias=b2), cls)


def _sc_gather(table, idx, *, e_pad, n_pad):
    from jax.experimental.pallas import tpu_sc as plsc

    B = 64
    e_per_core = e_pad // 2
    e_per_tile = e_per_core // 16
    nb = e_per_tile // B
    mesh = plsc.VectorSubcoreMesh(core_axis_name="c", subcore_axis_name="s")

    @functools.partial(
        pl.kernel, mesh=mesh,
        out_type=jax.ShapeDtypeStruct((e_pad, 128), jnp.float32),
        scratch_types=[
            pltpu.MemorySpace.VMEM((B,), jnp.int32),
            pltpu.MemorySpace.VMEM((B, 128), jnp.float32),
            pltpu.SemaphoreType.DMA,
        ],
    )
    def k(table_h, idx_h, out_h, sbuf, gbuf, sem):
        cid = lax.axis_index("c")
        sid = lax.axis_index("s")
        base0 = cid * e_per_core + sid * e_per_tile

        def batch(b, _):
            ebase = base0 + b * B
            pltpu.sync_copy(idx_h.at[pl.ds(ebase, B)], sbuf)
            pltpu.async_copy(table_h.at[sbuf], gbuf, sem).wait()
            pltpu.sync_copy(gbuf, out_h.at[pl.ds(ebase, B)])
            return ()
        lax.fori_loop(0, nb, batch, ())

    return k(table, idx)


def _sc_scatter_add(msg, idx, fill, *, e_pad, n_pad, slab_rows,
                    ones_mode=False):
    from jax.experimental.pallas import tpu_sc as plsc

    B = 64
    n_slabs = n_pad // slab_rows
    e_per_tile = e_pad // 16
    nb = e_per_tile // B
    acc_rows = _round_up(slab_rows + 1, 1024)
    nzc = acc_rows // 1024
    rpt = slab_rows // 16
    nwb = rpt // 16
    mesh = plsc.VectorSubcoreMesh(core_axis_name="c", subcore_axis_name="s")

    scratch = [
        pltpu.MemorySpace.VMEM_SHARED((acc_rows, 128), jnp.float32),
        pltpu.MemorySpace.VMEM((B,), jnp.int32),
        pltpu.MemorySpace.VMEM((B, 128), jnp.float32),
    ]

    @functools.partial(
        pl.kernel, mesh=mesh,
        out_type=jax.ShapeDtypeStruct((n_pad, 128), jnp.float32),
        scratch_types=scratch,
    )
    def k(*refs):
        if ones_mode:
            idx_h, fill_h, out_h, acc, dbuf, gbuf = refs
        else:
            msg_h, idx_h, fill_h, out_h, acc, dbuf, gbuf = refs
        cid = lax.axis_index("c")
        sid = lax.axis_index("s")

        if ones_mode:
            pltpu.sync_copy(fill_h.at[pl.ds(64, B)], gbuf)

        for j in range(n_slabs // 2):
            slab = j * 2 + cid
            lo = slab * slab_rows

            def zero_k(kk, _):
                pltpu.sync_copy(fill_h.at[pl.ds(0, 64)],
                                acc.at[pl.ds((sid + kk * 16) * 64, 64)])
                return ()
            lax.fori_loop(0, nzc, zero_k, ())
            plsc.subcore_barrier()

            def batch(b, _):
                ebase = sid * e_per_tile + b * B
                pltpu.sync_copy(idx_h.at[pl.ds(ebase, B)], dbuf)

                def li(i, _):
                    d = dbuf[pl.ds(i * 16, 16)] - lo
                    d = jnp.minimum(d, slab_rows)
                    neg = d >> 31
                    d = (d & ~neg) | (slab_rows & neg)
                    dbuf[pl.ds(i * 16, 16)] = d
                    return ()
                lax.fori_loop(0, B // 16, li, ())

                if not ones_mode:
                    pltpu.sync_copy(msg_h.at[pl.ds(ebase, B)], gbuf)
                pltpu.sync_copy(gbuf, acc.at[dbuf], add=True)
                return ()
            lax.fori_loop(0, nb, batch, ())
            plsc.subcore_barrier()

            def wb_k(kk, _):
                r = sid * rpt + kk * 16
                pltpu.sync_copy(acc.at[pl.ds(r, 16)],
                                out_h.at[pl.ds(lo + r, 16)])
                return ()
            lax.fori_loop(0, nwb, wb_k, ())
            plsc.subcore_barrier()

    if ones_mode:
        return k(idx, fill)
    return k(msg, idx, fill)


def _round_up(x, m):
    return ((x + m - 1) // m) * m


def _pick_slab(n_pad):
    n_slabs = 2
    while n_pad // n_slabs > 12544 or (n_pad // n_slabs) % 256:
        n_slabs *= 2
    return n_pad // n_slabs


def _rowmul_kernel(a_ref, w_ref, o_ref):
    o_ref[...] = a_ref[...] * w_ref[...]


def _rowmul(A, w, bn=1024):
    n, m = A.shape
    return pl.pallas_call(
        _rowmul_kernel,
        grid=(n // bn,),
        in_specs=[pl.BlockSpec((bn, m), lambda i: (i, 0)),
                  pl.BlockSpec((bn, 1), lambda i: (i, 0))],
        out_specs=pl.BlockSpec((bn, m), lambda i: (i, 0)),
        out_shape=jax.ShapeDtypeStruct((n, m), jnp.float32),
    )(A, w.reshape(n, 1))


def _edge_prep(src, dst, ew, n, e_pad):
    e = src.shape[0]
    pad = e_pad - e
    srcp = jnp.concatenate([src.astype(jnp.int32),
                            jnp.full((pad,), n, jnp.int32)])
    dstp = jnp.concatenate([dst.astype(jnp.int32),
                            jnp.full((pad,), n, jnp.int32)])
    ewp = jnp.pad(ew, (0, pad))
    fill = jnp.concatenate([jnp.zeros((64, 128), jnp.float32),
                            jnp.ones((64, 128), jnp.float32)], axis=0)
    return srcp, dstp, ewp, fill


def _seg_aggregate_sc(h, srcp, dstp, ewp, fill, n, n_pad, e_pad, slab_rows):
    outs = []
    for c0 in range(0, h.shape[1], 128):
        table = jnp.pad(h[:, c0:c0 + 128], ((0, n_pad - n), (0, 0)))
        gath = _sc_gather(table, srcp, e_pad=e_pad, n_pad=n_pad)
        msg = _rowmul(gath, ewp)
        outs.append(_sc_scatter_add(msg, dstp, fill, e_pad=e_pad,
                                    n_pad=n_pad, slab_rows=slab_rows)[:n])
    return jnp.concatenate(outs, axis=1) if len(outs) > 1 else outs[0]


def _degrees_sc(srcp, dstp, fill, n, n_pad, e_pad, slab_rows):
    dego = _sc_scatter_add(None, srcp, fill, e_pad=e_pad, n_pad=n_pad,
                           slab_rows=slab_rows, ones_mode=True)[:n, 0]
    degi = _sc_scatter_add(None, dstp, fill, e_pad=e_pad, n_pad=n_pad,
                           slab_rows=slab_rows, ones_mode=True)[:n, 0]
    return dego, degi


def kernel(patch_feats, patch_edge_weights, mesh_edge_weights, params,
           patch_edge_index, mesh_edge_index):
    p = params
    NPN = patch_feats.shape[0]
    P = NPN // 10
    ps, pd = patch_edge_index[0], patch_edge_index[1]
    ms, md = mesh_edge_index[0], mesh_edge_index[1]

    np_pad = _round_up(NPN + 1, 2048)
    ep_pad = _round_up(ps.shape[0], 2048)
    p_slab = _pick_slab(np_pad)
    psp, pdp, pewp, fillp = _edge_prep(ps, pd, patch_edge_weights, NPN, ep_pad)
    deg_out_p, deg_in_p = _degrees_sc(psp, pdp, fillp, NPN, np_pad, ep_pad,
                                      p_slab)

    xs1 = _mm(patch_feats, scale=deg_out_p, epi='none')
    agg1 = _seg_aggregate_sc(xs1, psp, pdp, pewp, fillp, NPN, np_pad, ep_pad,
                             p_slab)
    h1 = _mm(agg1, p['pe_conv1_W'], scale=deg_in_p,
             aux=_aux_pack(p['pe_conv1_W'].shape[1], alpha=p['pe_gn1_alpha'],
                           gamma=p['pe_gn1_gamma'], beta=p['pe_gn1_beta']),
             epi='gn')

    s1p = _mm(h1, p['pe_ro1_phi_W'],
              aux=_aux_pack(p['pe_ro1_phi_W'].shape[1], bias=p['pe_ro1_phi_b']),
              epi='bias_leaky_segsum10')
    ro1 = _mm(s1p, p['pe_ro1_rho_W'],
              aux=_aux_pack(p['pe_ro1_rho_W'].shape[1], bias=p['pe_ro1_rho_b']),
              epi='bias_leaky')
    m1 = _mm(h1, epi='segsum10', post_scale=0.1)

    g2 = _mm(h1, p['pe_conv2_W'], scale=deg_out_p, epi='none')
    agg2 = _seg_aggregate_sc(g2, psp, pdp, pewp, fillp, NPN, np_pad, ep_pad,
                             p_slab)
    h2 = _mm(agg2, scale=deg_in_p,
             aux=_aux_pack(agg2.shape[1], alpha=p['pe_gn2_alpha'],
                           gamma=p['pe_gn2_gamma'], beta=p['pe_gn2_beta']),
             epi='gn')

    s2p = _mm(h2, p['pe_ro2_phi_W'],
              aux=_aux_pack(p['pe_ro2_phi_W'].shape[1], bias=p['pe_ro2_phi_b']),
              epi='bias_leaky_segsum10')
    ro2 = _mm(s2p, p['pe_ro2_rho_W'],
              aux=_aux_pack(p['pe_ro2_rho_W'].shape[1], bias=p['pe_ro2_rho_b']),
              epi='bias_leaky')
    m2 = _mm(h2, epi='segsum10', post_scale=0.1)

    cat = jnp.concatenate([ro1, m1, ro2, m2], axis=1)
    readouts = _mm(cat, p['pe_emb_W'], epi='inorm')

    NM = readouts.shape[0]
    nm_pad = _round_up(NM + 1, 2048)
    em_pad = _round_up(ms.shape[0], 2048)
    m_slab = _pick_slab(nm_pad)
    msp, mdp, mewp, fillm = _edge_prep(ms, md, mesh_edge_weights, NM, em_pad)
    deg_out_m, deg_in_m = _degrees_sc(msp, mdp, fillm, NM, nm_pad, em_pad,
                                      m_slab)

    xsm1 = _mm(readouts, scale=deg_out_m, epi='none')
    aggm1 = _seg_aggregate_sc(xsm1, msp, mdp, mewp, fillm, NM, nm_pad, em_pad,
                              m_slab)
    t1 = _mm(aggm1, p['mr_conv1_W'], scale=deg_in_m, epi='none')
    stats1 = _mm(t1, epi='colstats')
    auxg1 = _aux_pack(t1.shape[1], alpha=p['mr_gn1_alpha'],
                      gamma=p['mr_gn1_gamma'], beta=p['mr_gn1_beta'])
    u1 = _mm(t1, aux=auxg1, stats=stats1, epi='gn_global')

    s1m = _mm(u1, p['mr_ro1_phi_W'],
              aux=_aux_pack(p['mr_ro1_phi_W'].shape[1], bias=p['mr_ro1_phi_b']),
              epi='bias_leaky_colsum')

    xsm2 = _mm(u1, scale=deg_out_m, epi='none')
    aggm2 = _seg_aggregate_sc(xsm2, msp, mdp, mewp, fillm, NM, nm_pad, em_pad,
                              m_slab)
    t2 = _mm(aggm2, p['mr_conv2_W'], scale=deg_in_m, epi='none')
    stats2 = _mm(t2, epi='colstats')
    auxg2 = _aux_pack(t2.shape[1], alpha=p['mr_gn2_alpha'],
                      gamma=p['mr_gn2_gamma'], beta=p['mr_gn2_beta'])
    u2 = _mm(t2, aux=auxg2, stats=stats2, epi='gn_global')

    s2m = _mm(u2, p['mr_ro2_phi_W'],
              aux=_aux_pack(p['mr_ro2_phi_W'].shape[1], bias=p['mr_ro2_phi_b']),
              epi='bias_leaky_colsum')

    out = _tail(s1m, s2m, p['mr_ro1_rho_W'], p['mr_ro1_rho_b'],
                p['mr_ro2_rho_W'], p['mr_ro2_rho_b'], p['mr_cls_W'])
    return out[0:1, :]


# speedup vs baseline: 1.6341x; 1.1599x over previous
import functools

import jax
import jax.numpy as jnp
from jax import lax
from jax.experimental import pallas as pl
from jax.experimental.pallas import tpu as pltpu

NEG = 0.01
_EPS = 1e-5


def _leaky(x):
    return jnp.where(x >= 0, x, NEG * x)


def _mm_kernel_body(refs, *, epi, has_w, has_aux, has_scale, has_stats, bn,
                    nrows_total, post_scale):
    idx = 0
    a_ref = refs[idx]; idx += 1
    w_ref = None
    if has_w:
        w_ref = refs[idx]; idx += 1
    aux_ref = None
    if has_aux:
        aux_ref = refs[idx]; idx += 1
    scale_ref = None
    if has_scale:
        scale_ref = refs[idx]; idx += 1
    stats_ref = None
    if has_stats:
        stats_ref = refs[idx]; idx += 1
    o_ref = refs[idx]

    a = a_ref[...]
    if has_scale:
        s = lax.rsqrt(jnp.maximum(scale_ref[...], 1.0))
        a = a * s
    if has_w:
        y = jnp.dot(a, w_ref[...], preferred_element_type=jnp.float32)
    else:
        y = a

    if epi == 'none':
        o_ref[...] = y
    elif epi == 'bias_leaky':
        y = _leaky(y + aux_ref[0, :][None, :])
        o_ref[...] = y
    elif epi == 'gn':
        m = y.shape[1]
        al = aux_ref[1, :][None, :]
        ga = aux_ref[2, :][None, :]
        be = aux_ref[3, :][None, :]
        r = y.reshape(bn // 10, 10, m)
        mean = jnp.mean(r, axis=1)
        xc = r - (al * mean)[:, None, :]
        var = jnp.mean(xc * xc, axis=1)
        out = _leaky(ga[:, None, :] * xc / jnp.sqrt(var[:, None, :] + _EPS)
                     + be[:, None, :])
        o_ref[...] = out.reshape(bn, m)
    elif epi == 'gn_global':
        n = float(nrows_total)
        s1 = stats_ref[0, :][None, :]
        s2 = stats_ref[1, :][None, :]
        mean = s1 / n
        ex2 = s2 / n
        al = aux_ref[1, :][None, :]
        ga = aux_ref[2, :][None, :]
        be = aux_ref[3, :][None, :]
        var = ex2 - 2.0 * al * mean * mean + al * al * mean * mean
        xc = y - al * mean
        o_ref[...] = _leaky(ga * xc / jnp.sqrt(var + _EPS) + be)
    elif epi == 'inorm':
        m = jnp.mean(y, axis=1, keepdims=True)
        v = jnp.mean((y - m) * (y - m), axis=1, keepdims=True)
        o_ref[...] = _leaky((y - m) / jnp.sqrt(v + _EPS))
    elif epi == 'segsum10':
        m = y.shape[1]
        r = y.reshape(bn // 10, 10, m).sum(axis=1)
        if post_scale != 1.0:
            r = r * post_scale
        o_ref[...] = r
    elif epi == 'bias_leaky_segsum10':
        m = y.shape[1]
        y = _leaky(y + aux_ref[0, :][None, :])
        r = y.reshape(bn // 10, 10, m).sum(axis=1)
        if post_scale != 1.0:
            r = r * post_scale
        o_ref[...] = r
    elif epi == 'colstats':
        m = y.shape[1]
        s1 = jnp.sum(y, axis=0, keepdims=True)
        s2 = jnp.sum(y * y, axis=0, keepdims=True)
        blk = jnp.concatenate([s1, s2, jnp.zeros((6, m), jnp.float32)], axis=0)

        @pl.when(pl.program_id(0) == 0)
        def _():
            o_ref[...] = blk

        @pl.when(pl.program_id(0) != 0)
        def _():
            o_ref[...] = o_ref[...] + blk
    elif epi == 'bias_leaky_colsum':
        m = y.shape[1]
        y = _leaky(y + aux_ref[0, :][None, :])
        s1 = jnp.sum(y, axis=0, keepdims=True)
        blk = jnp.concatenate([s1, jnp.zeros((7, m), jnp.float32)], axis=0)

        @pl.when(pl.program_id(0) == 0)
        def _():
            o_ref[...] = blk

        @pl.when(pl.program_id(0) != 0)
        def _():
            o_ref[...] = o_ref[...] + blk
    else:
        raise ValueError(epi)


def _mm(A, W=None, aux=None, scale=None, stats=None, *, epi='none',
        bn=2000, post_scale=1.0):
    n, k = A.shape
    assert n % bn == 0, (n, bn)
    m = W.shape[1] if W is not None else k
    grid = (n // bn,)

    in_specs = [pl.BlockSpec((bn, k), lambda i: (i, 0))]
    args = [A]
    if W is not None:
        in_specs.append(pl.BlockSpec((k, m), lambda i: (0, 0)))
        args.append(W)
    if aux is not None:
        in_specs.append(pl.BlockSpec((8, m), lambda i: (0, 0)))
        args.append(aux)
    if scale is not None:
        in_specs.append(pl.BlockSpec((bn, 1), lambda i: (i, 0)))
        args.append(scale.reshape(n, 1))
    if stats is not None:
        in_specs.append(pl.BlockSpec((8, m), lambda i: (0, 0)))
        args.append(stats)

    if epi in ('segsum10', 'bias_leaky_segsum10'):
        out_shape = jax.ShapeDtypeStruct((n // 10, m), jnp.float32)
        out_spec = pl.BlockSpec((bn // 10, m), lambda i: (i, 0))
    elif epi in ('colstats', 'bias_leaky_colsum'):
        out_shape = jax.ShapeDtypeStruct((8, m), jnp.float32)
        out_spec = pl.BlockSpec((8, m), lambda i: (0, 0))
    else:
        out_shape = jax.ShapeDtypeStruct((n, m), jnp.float32)
        out_spec = pl.BlockSpec((bn, m), lambda i: (i, 0))

    body = functools.partial(
        _mm_kernel_body, epi=epi, has_w=W is not None, has_aux=aux is not None,
        has_scale=scale is not None, has_stats=stats is not None, bn=bn,
        nrows_total=n, post_scale=post_scale)

    def kern(*refs):
        _mm_kernel_body_wrapper(refs, body)

    return pl.pallas_call(
        kern,
        grid=grid,
        in_specs=in_specs,
        out_specs=out_spec,
        out_shape=out_shape,
    )(*args)


def _mm_kernel_body_wrapper(refs, body):
    body(refs)


def _aux_pack(m, bias=None, alpha=None, gamma=None, beta=None):
    rows = []
    for v in (bias, alpha, gamma, beta):
        rows.append(jnp.zeros((m,), jnp.float32) if v is None else v)
    rows += [jnp.zeros((m,), jnp.float32)] * 4
    return jnp.stack(rows, axis=0)


def _tail_kernel(s1_ref, s2_ref, w1_ref, b1_ref, w2_ref, b2_ref, cls_ref,
                 o_ref):
    r1 = _leaky(jnp.dot(s1_ref[...], w1_ref[...],
                        preferred_element_type=jnp.float32) + b1_ref[0, :][None, :])
    r2 = _leaky(jnp.dot(s2_ref[...], w2_ref[...],
                        preferred_element_type=jnp.float32) + b2_ref[0, :][None, :])
    cat = jnp.concatenate([r1, r2], axis=1)
    o_ref[...] = jnp.dot(cat, cls_ref[...], preferred_element_type=jnp.float32)


def _tail(s1, s2, w1, b1, w2, b2, cls):
    kq = w1.shape[0]
    mq = w1.shape[1]
    out_n = cls.shape[1]
    return pl.pallas_call(
        _tail_kernel,
        out_shape=jax.ShapeDtypeStruct((8, out_n), jnp.float32),
    )(s1, s2, w1, _aux_pack(mq, bias=b1), w2, _aux_pack(mq, bias=b2), cls)


def _sc_gather(table, idx, *, e_pad, n_pad):
    from jax.experimental.pallas import tpu_sc as plsc

    B = 128
    e_per_core = e_pad // 2
    e_per_tile = e_per_core // 16
    nb = e_per_tile // B
    mesh = plsc.VectorSubcoreMesh(core_axis_name="c", subcore_axis_name="s")

    @functools.partial(
        pl.kernel, mesh=mesh,
        out_type=jax.ShapeDtypeStruct((e_pad, 128), jnp.float32),
        scratch_types=[
            pltpu.MemorySpace.VMEM((B,), jnp.int32),
            pltpu.MemorySpace.VMEM((B, 128), jnp.float32),
            pltpu.SemaphoreType.DMA,
        ],
    )
    def k(table_h, idx_h, out_h, sbuf, gbuf, sem):
        cid = lax.axis_index("c")
        sid = lax.axis_index("s")
        base0 = cid * e_per_core + sid * e_per_tile

        def batch(b, _):
            ebase = base0 + b * B
            pltpu.sync_copy(idx_h.at[pl.ds(ebase, B)], sbuf)
            pltpu.async_copy(table_h.at[sbuf], gbuf, sem).wait()
            pltpu.sync_copy(gbuf, out_h.at[pl.ds(ebase, B)])
            return ()
        lax.fori_loop(0, nb, batch, ())

    return k(table, idx)


def _sc_scatter_add(msg, idx, fill, *, e_pad, n_pad, slab_rows,
                    ones_mode=False):
    from jax.experimental.pallas import tpu_sc as plsc

    B = 128
    n_slabs = n_pad // slab_rows
    e_per_tile = e_pad // 16
    nb = e_per_tile // B
    acc_rows = _round_up(slab_rows + 1, 1024)
    nzc = acc_rows // 1024
    rpt = slab_rows // 16
    nwb = rpt // 16
    mesh = plsc.VectorSubcoreMesh(core_axis_name="c", subcore_axis_name="s")

    scratch = [
        pltpu.MemorySpace.VMEM_SHARED((acc_rows, 128), jnp.float32),
        pltpu.MemorySpace.VMEM((B,), jnp.int32),
        pltpu.MemorySpace.VMEM((B, 128), jnp.float32),
    ]

    @functools.partial(
        pl.kernel, mesh=mesh,
        out_type=jax.ShapeDtypeStruct((n_pad, 128), jnp.float32),
        scratch_types=scratch,
    )
    def k(*refs):
        if ones_mode:
            idx_h, fill_h, out_h, acc, dbuf, gbuf = refs
        else:
            msg_h, idx_h, fill_h, out_h, acc, dbuf, gbuf = refs
        cid = lax.axis_index("c")
        sid = lax.axis_index("s")

        if ones_mode:
            pltpu.sync_copy(fill_h.at[pl.ds(64, 64)], gbuf.at[pl.ds(0, 64)])
            pltpu.sync_copy(fill_h.at[pl.ds(64, 64)], gbuf.at[pl.ds(64, 64)])

        for j in range(n_slabs // 2):
            slab = j * 2 + cid
            lo = slab * slab_rows

            def zero_k(kk, _):
                pltpu.sync_copy(fill_h.at[pl.ds(0, 64)],
                                acc.at[pl.ds((sid + kk * 16) * 64, 64)])
                return ()
            lax.fori_loop(0, nzc, zero_k, ())
            plsc.subcore_barrier()

            def batch(b, _):
                ebase = sid * e_per_tile + b * B
                pltpu.sync_copy(idx_h.at[pl.ds(ebase, B)], dbuf)

                def li(i, _):
                    d = dbuf[pl.ds(i * 16, 16)] - lo
                    d = jnp.minimum(d, slab_rows)
                    neg = d >> 31
                    d = (d & ~neg) | (slab_rows & neg)
                    dbuf[pl.ds(i * 16, 16)] = d
                    return ()
                lax.fori_loop(0, B // 16, li, ())

                if not ones_mode:
                    pltpu.sync_copy(msg_h.at[pl.ds(ebase, B)], gbuf)
                pltpu.sync_copy(gbuf, acc.at[dbuf], add=True)
                return ()
            lax.fori_loop(0, nb, batch, ())
            plsc.subcore_barrier()

            def wb_k(kk, _):
                r = sid * rpt + kk * 16
                pltpu.sync_copy(acc.at[pl.ds(r, 16)],
                                out_h.at[pl.ds(lo + r, 16)])
                return ()
            lax.fori_loop(0, nwb, wb_k, ())
            plsc.subcore_barrier()

    if ones_mode:
        return k(idx, fill)
    return k(msg, idx, fill)


def _round_up(x, m):
    return ((x + m - 1) // m) * m


def _pick_slab(n_pad):
    n_slabs = 2
    while n_pad // n_slabs > 12544 or (n_pad // n_slabs) % 256:
        n_slabs *= 2
    return n_pad // n_slabs


def _rowmul_kernel(a_ref, w_ref, o_ref):
    o_ref[...] = a_ref[...] * w_ref[...]


def _rowmul(A, w, bn=1024):
    n, m = A.shape
    return pl.pallas_call(
        _rowmul_kernel,
        grid=(n // bn,),
        in_specs=[pl.BlockSpec((bn, m), lambda i: (i, 0)),
                  pl.BlockSpec((bn, 1), lambda i: (i, 0))],
        out_specs=pl.BlockSpec((bn, m), lambda i: (i, 0)),
        out_shape=jax.ShapeDtypeStruct((n, m), jnp.float32),
    )(A, w.reshape(n, 1))


def _edge_prep(src, dst, ew, n, e_pad):
    e = src.shape[0]
    pad = e_pad - e
    srcp = jnp.concatenate([src.astype(jnp.int32),
                            jnp.full((pad,), n, jnp.int32)])
    dstp = jnp.concatenate([dst.astype(jnp.int32),
                            jnp.full((pad,), n, jnp.int32)])
    ewp = jnp.pad(ew, (0, pad))
    fill = jnp.concatenate([jnp.zeros((64, 128), jnp.float32),
                            jnp.ones((64, 128), jnp.float32)], axis=0)
    return srcp, dstp, ewp, fill


def _seg_aggregate_sc(h, srcp, dstp, ewp, fill, n, n_pad, e_pad, slab_rows):
    outs = []
    for c0 in range(0, h.shape[1], 128):
        table = jnp.pad(h[:, c0:c0 + 128], ((0, n_pad - n), (0, 0)))
        gath = _sc_gather(table, srcp, e_pad=e_pad, n_pad=n_pad)
        msg = _rowmul(gath, ewp)
        outs.append(_sc_scatter_add(msg, dstp, fill, e_pad=e_pad,
                                    n_pad=n_pad, slab_rows=slab_rows)[:n])
    return jnp.concatenate(outs, axis=1) if len(outs) > 1 else outs[0]


def _degrees_sc(srcp, dstp, fill, n, n_pad, e_pad, slab_rows):
    dego = _sc_scatter_add(None, srcp, fill, e_pad=e_pad, n_pad=n_pad,
                           slab_rows=slab_rows, ones_mode=True)[:n, 0]
    degi = _sc_scatter_add(None, dstp, fill, e_pad=e_pad, n_pad=n_pad,
                           slab_rows=slab_rows, ones_mode=True)[:n, 0]
    return dego, degi


def kernel(patch_feats, patch_edge_weights, mesh_edge_weights, params,
           patch_edge_index, mesh_edge_index):
    p = params
    NPN = patch_feats.shape[0]
    P = NPN // 10
    ps, pd = patch_edge_index[0], patch_edge_index[1]
    ms, md = mesh_edge_index[0], mesh_edge_index[1]

    np_pad = _round_up(NPN + 1, 2048)
    ep_pad = _round_up(ps.shape[0], 2048)
    p_slab = _pick_slab(np_pad)
    psp, pdp, pewp, fillp = _edge_prep(ps, pd, patch_edge_weights, NPN, ep_pad)
    deg_out_p, deg_in_p = _degrees_sc(psp, pdp, fillp, NPN, np_pad, ep_pad,
                                      p_slab)

    xs1 = _mm(patch_feats, scale=deg_out_p, epi='none')
    agg1 = _seg_aggregate_sc(xs1, psp, pdp, pewp, fillp, NPN, np_pad, ep_pad,
                             p_slab)
    h1 = _mm(agg1, p['pe_conv1_W'], scale=deg_in_p,
             aux=_aux_pack(p['pe_conv1_W'].shape[1], alpha=p['pe_gn1_alpha'],
                           gamma=p['pe_gn1_gamma'], beta=p['pe_gn1_beta']),
             epi='gn')

    s1p = _mm(h1, p['pe_ro1_phi_W'],
              aux=_aux_pack(p['pe_ro1_phi_W'].shape[1], bias=p['pe_ro1_phi_b']),
              epi='bias_leaky_segsum10')
    ro1 = _mm(s1p, p['pe_ro1_rho_W'],
              aux=_aux_pack(p['pe_ro1_rho_W'].shape[1], bias=p['pe_ro1_rho_b']),
              epi='bias_leaky')
    m1 = _mm(h1, epi='segsum10', post_scale=0.1)

    g2 = _mm(h1, p['pe_conv2_W'], scale=deg_out_p, epi='none')
    agg2 = _seg_aggregate_sc(g2, psp, pdp, pewp, fillp, NPN, np_pad, ep_pad,
                             p_slab)
    h2 = _mm(agg2, scale=deg_in_p,
             aux=_aux_pack(agg2.shape[1], alpha=p['pe_gn2_alpha'],
                           gamma=p['pe_gn2_gamma'], beta=p['pe_gn2_beta']),
             epi='gn')

    s2p = _mm(h2, p['pe_ro2_phi_W'],
              aux=_aux_pack(p['pe_ro2_phi_W'].shape[1], bias=p['pe_ro2_phi_b']),
              epi='bias_leaky_segsum10')
    ro2 = _mm(s2p, p['pe_ro2_rho_W'],
              aux=_aux_pack(p['pe_ro2_rho_W'].shape[1], bias=p['pe_ro2_rho_b']),
              epi='bias_leaky')
    m2 = _mm(h2, epi='segsum10', post_scale=0.1)

    cat = jnp.concatenate([ro1, m1, ro2, m2], axis=1)
    readouts = _mm(cat, p['pe_emb_W'], epi='inorm')

    NM = readouts.shape[0]
    nm_pad = _round_up(NM + 1, 2048)
    em_pad = _round_up(ms.shape[0], 2048)
    m_slab = _pick_slab(nm_pad)
    msp, mdp, mewp, fillm = _edge_prep(ms, md, mesh_edge_weights, NM, em_pad)
    deg_out_m, deg_in_m = _degrees_sc(msp, mdp, fillm, NM, nm_pad, em_pad,
                                      m_slab)

    xsm1 = _mm(readouts, scale=deg_out_m, epi='none')
    aggm1 = _seg_aggregate_sc(xsm1, msp, mdp, mewp, fillm, NM, nm_pad, em_pad,
                              m_slab)
    t1 = _mm(aggm1, p['mr_conv1_W'], scale=deg_in_m, epi='none')
    stats1 = _mm(t1, epi='colstats')
    auxg1 = _aux_pack(t1.shape[1], alpha=p['mr_gn1_alpha'],
                      gamma=p['mr_gn1_gamma'], beta=p['mr_gn1_beta'])
    u1 = _mm(t1, aux=auxg1, stats=stats1, epi='gn_global')

    s1m = _mm(u1, p['mr_ro1_phi_W'],
              aux=_aux_pack(p['mr_ro1_phi_W'].shape[1], bias=p['mr_ro1_phi_b']),
              epi='bias_leaky_colsum')

    xsm2 = _mm(u1, scale=deg_out_m, epi='none')
    aggm2 = _seg_aggregate_sc(xsm2, msp, mdp, mewp, fillm, NM, nm_pad, em_pad,
                              m_slab)
    t2 = _mm(aggm2, p['mr_conv2_W'], scale=deg_in_m, epi='none')
    stats2 = _mm(t2, epi='colstats')
    auxg2 = _aux_pack(t2.shape[1], alpha=p['mr_gn2_alpha'],
                      gamma=p['mr_gn2_gamma'], beta=p['mr_gn2_beta'])
    u2 = _mm(t2, aux=auxg2, stats=stats2, epi='gn_global')

    s2m = _mm(u2, p['mr_ro2_phi_W'],
              aux=_aux_pack(p['mr_ro2_phi_W'].shape[1], bias=p['mr_ro2_phi_b']),
              epi='bias_leaky_colsum')

    out = _tail(s1m, s2m, p['mr_ro1_rho_W'], p['mr_ro1_rho_b'],
                p['mr_ro2_rho_W'], p['mr_ro2_rho_b'], p['mr_cls_W'])
    return out[0:1, :]
